# Initial kernel scaffold; baseline (speedup 1.0000x reference)
import jax
import jax.numpy as jnp
from jax.experimental import pallas as pl


def kernel(x, cluster, edge_index, identifier, valid_len, time_step_len, params):
    raise NotImplementedError("write your pallas kernel here")


# speedup vs baseline: 1.2609x; 1.2609x over previous
import functools

import jax
import jax.numpy as jnp
import numpy as np
from jax import lax
from jax.experimental import pallas as pl
from jax.experimental.pallas import tpu as pltpu
from jax.experimental.pallas import tpu_sc as plsc

_IN_CH = 10
_HID = 64
_NL = 3
_B = 64
_T = 256
_NC = _B * _T
_NN = 327680
_BLK = 512
_NBLK = _NN // _BLK


def _ln(x, g, b, eps=1e-5):
    ones = jnp.full((_HID, _HID), 1.0 / _HID, jnp.float32)
    mu = jnp.dot(x, ones, preferred_element_type=jnp.float32)
    m2 = jnp.dot(x * x, ones, preferred_element_type=jnp.float32)
    var = m2 - mu * mu
    return (x - mu) * jax.lax.rsqrt(var + eps) * g + b


def _l0_body(x_ref, wc_ref, bc_ref, w2_ref, b2_ref, g1_ref, be1_ref,
             g2_ref, be2_ref, o_ref):
    xb = x_ref[...]
    hc = jnp.dot(xb, wc_ref[...], preferred_element_type=jnp.float32) + bc_ref[...]
    h1 = hc[:, :_HID]
    hs = hc[:, _HID:]
    t = jax.nn.relu(_ln(h1, g1_ref[...], be1_ref[...]))
    u = jnp.dot(t, w2_ref[...], preferred_element_type=jnp.float32) + b2_ref[...]
    o_ref[...] = jax.nn.relu(_ln(u, g2_ref[...], be2_ref[...]) + hs)


def _layer0(x, p):
    wc = jnp.concatenate([p['W1'], p['Ws']], axis=1)
    bc = jnp.concatenate([p['b1'], p['bs']])[None, :]
    full = lambda shape: pl.BlockSpec(shape, lambda i: tuple(0 for _ in shape))
    return pl.pallas_call(
        _l0_body,
        grid=(_NBLK,),
        in_specs=[
            pl.BlockSpec((_BLK, _IN_CH), lambda i: (i, 0)),
            full((_IN_CH, 2 * _HID)), full((1, 2 * _HID)),
            full((_HID, _HID)), full((1, _HID)),
            full((1, _HID)), full((1, _HID)), full((1, _HID)), full((1, _HID)),
        ],
        out_specs=pl.BlockSpec((_BLK, _HID), lambda i: (i, 0)),
        out_shape=jax.ShapeDtypeStruct((_NN, _HID), jnp.float32),
    )(x, wc, bc, p['W2'], p['b2'][None, :], p['g1'][None, :], p['be1'][None, :],
      p['g2'][None, :], p['be2'][None, :])


def _l12_body(extra_y, xn_ref, ag_ref, wcn_ref, wca_ref, bc_ref, w2_ref,
              b2_ref, g1_ref, be1_ref, g2_ref, be2_ref, *rest):
    if extra_y:
        wl_ref, o_ref, y_ref = rest
    else:
        (o_ref,) = rest
    xn = xn_ref[...]
    ag = ag_ref[...]
    hc = (jnp.dot(xn, wcn_ref[...], preferred_element_type=jnp.float32)
          + jnp.dot(ag, wca_ref[...], preferred_element_type=jnp.float32)
          + bc_ref[...])
    h1 = hc[:, :_HID]
    hs = hc[:, _HID:]
    t = jax.nn.relu(_ln(h1, g1_ref[...], be1_ref[...]))
    u = jnp.dot(t, w2_ref[...], preferred_element_type=jnp.float32) + b2_ref[...]
    out = jax.nn.relu(_ln(u, g2_ref[...], be2_ref[...]) + hs)
    o_ref[...] = out
    if extra_y:
        y_ref[...] = jnp.dot(out, wl_ref[...], preferred_element_type=jnp.float32)


def _layer12(xn, ag, p, wla=None):
    wcn = jnp.concatenate([p['W1'][:_HID], p['Ws'][:_HID]], axis=1)
    wca = jnp.concatenate([p['W1'][_HID:], p['Ws'][_HID:]], axis=1)
    bc = jnp.concatenate([p['b1'], p['bs']])[None, :]
    full = lambda shape: pl.BlockSpec(shape, lambda i: tuple(0 for _ in shape))
    extra = wla is not None
    in_specs = [
        pl.BlockSpec((_BLK, _HID), lambda i: (i, 0)),
        pl.BlockSpec((_BLK, _HID), lambda i: (i, 0)),
        full((_HID, 2 * _HID)), full((_HID, 2 * _HID)), full((1, 2 * _HID)),
        full((_HID, _HID)), full((1, _HID)),
        full((1, _HID)), full((1, _HID)), full((1, _HID)), full((1, _HID)),
    ]
    args = [xn, ag, wcn, wca, bc, p['W2'], p['b2'][None, :], p['g1'][None, :],
            p['be1'][None, :], p['g2'][None, :], p['be2'][None, :]]
    if extra:
        in_specs.append(full((_HID, _HID)))
        args.append(wla)
        out_specs = (pl.BlockSpec((_BLK, _HID), lambda i: (i, 0)),
                     pl.BlockSpec((_BLK, _HID), lambda i: (i, 0)))
        out_shape = (jax.ShapeDtypeStruct((_NN, _HID), jnp.float32),
                     jax.ShapeDtypeStruct((_NN, _HID), jnp.float32))
    else:
        out_specs = pl.BlockSpec((_BLK, _HID), lambda i: (i, 0))
        out_shape = jax.ShapeDtypeStruct((_NN, _HID), jnp.float32)
    return pl.pallas_call(
        functools.partial(_l12_body, extra),
        grid=(_NBLK,),
        in_specs=in_specs,
        out_specs=out_specs,
        out_shape=out_shape,
    )(*args)


def _fin_body(ac_ref, wlb_ref, bl_ref, o_ref):
    a3 = ac_ref[:, :_HID]
    m = ac_ref[:, _HID:]
    present = m[:, :1] > -1e30
    sr = m + jnp.dot(a3, wlb_ref[...], preferred_element_type=jnp.float32) + bl_ref[...]
    sub = jnp.where(present, sr, 0.0)
    n = jnp.maximum(jnp.sqrt(jnp.sum(sub * sub, axis=-1, keepdims=True)), 1e-12)
    o_ref[...] = sub / n


def _finalize(aggcat, wlb, bl):
    full = lambda shape: pl.BlockSpec(shape, lambda i: tuple(0 for _ in shape))
    return pl.pallas_call(
        _fin_body,
        grid=(_NC // _BLK,),
        in_specs=[pl.BlockSpec((_BLK, 2 * _HID), lambda i: (i, 0)),
                  full((_HID, _HID)), full((1, _HID))],
        out_specs=pl.BlockSpec((_BLK, _HID), lambda i: (i, 0)),
        out_shape=jax.ShapeDtypeStruct((_NC, _HID), jnp.float32),
    )(aggcat, wlb, bl[None, :])


def _attn_body(vl_ref, f_ref, wq_ref, wk_ref, wv_ref, o_ref):
    b = pl.program_id(0)
    f = f_ref[0]
    q = jnp.dot(f, wq_ref[...], preferred_element_type=jnp.float32)
    k = jnp.dot(f, wk_ref[...], preferred_element_type=jnp.float32)
    v = jnp.dot(f, wv_ref[...], preferred_element_type=jnp.float32)
    s = jax.lax.dot_general(q, k, (((1,), (1,)), ((), ())),
                            preferred_element_type=jnp.float32)
    vl = vl_ref[b]
    col = jax.lax.broadcasted_iota(jnp.int32, (_T, _T), 1)
    s = jnp.where(col >= vl, -1e12, s)
    s = s - jnp.max(s, axis=-1, keepdims=True)
    e = jnp.exp(s)
    a = e / jnp.sum(e, axis=-1, keepdims=True)
    o_ref[0] = jnp.dot(a, v, preferred_element_type=jnp.float32)


def _attention(feats_pad, vl, wq, wk, wv):
    return pl.pallas_call(
        _attn_body,
        grid=(_B,),
        in_specs=[
            pl.BlockSpec(memory_space=pltpu.SMEM),
            pl.BlockSpec((1, _T, 128), lambda b: (b, 0, 0)),
            pl.BlockSpec((128, _HID), lambda b: (0, 0)),
            pl.BlockSpec((128, _HID), lambda b: (0, 0)),
            pl.BlockSpec((128, _HID), lambda b: (0, 0)),
        ],
        out_specs=pl.BlockSpec((1, _T, _HID), lambda b: (b, 0, 0)),
        out_shape=jax.ShapeDtypeStruct((_B, _T, _HID), jnp.float32),
    )(vl, feats_pad, wq, wk, wv)


_NW = 32
_CPT = _NC // _NW
_CH = 512
_CH2 = 256
_CROWS = _NN // 8
_DUMP = _NN


def _splat(x):
    return jnp.full((16,), x, jnp.int32)


def _sc_segmax_gather(vals, clus2d, starts):
    mesh = plsc.VectorSubcoreMesh(core_axis_name="c", subcore_axis_name="s")

    @functools.partial(
        pl.kernel, mesh=mesh,
        compiler_params=pltpu.CompilerParams(
            needs_layout_passes=False, use_tc_tiling_on_sc=False),
        out_type=jax.ShapeDtypeStruct((_NN + 8, _HID), jnp.float32),
        scratch_types=[
            pltpu.VMEM((40,), jnp.int32),
            pltpu.VMEM((72, 8), jnp.int32),
            pltpu.VMEM((_CH + 8, _HID), jnp.float32),
            pltpu.VMEM((_CPT + 1, _HID), jnp.float32),
            pltpu.VMEM((_CH, _HID), jnp.float32),
            pltpu.VMEM((4, 128), jnp.int32),
            pltpu.SemaphoreType.DMA,
        ],
    )
    def k(vals_hbm, clus_hbm, starts_hbm, out_hbm,
          starts_v, clus_v, vals_v, slice_v, outv_v, idx_v, sem):
        iota = lax.iota(jnp.int32, 16)
        wid = lax.axis_index("s") * 2 + lax.axis_index("c")
        pltpu.sync_copy(starts_hbm, starts_v)

        def rd(i):
            return jnp.max(plsc.load_gather(starts_v, [_splat(i)]))

        s0 = rd(wid)
        s1 = rd(wid + 1)
        nchunks = (s1 - s0 + _CH - 1) // _CH

        def init_body(r, _):
            z = jnp.zeros((16,), jnp.float32)
            for g in range(_HID // 16):
                slice_v[r, pl.ds(16 * g, 16)] = z
            return 0

        lax.fori_loop(0, _CPT + 1, init_body, 0)

        def load_clus(start):
            crow = jnp.minimum(8 * (start // 64), _CROWS - 72)
            pltpu.sync_copy(clus_hbm.at[pl.ds(crow, 72)], clus_v)
            return start - crow * 8

        def chunk_a(kk, carry):
            start = s0 + kk * _CH
            vstart = jnp.minimum(8 * (start // 8), _NN - _CH - 8)
            voff = start - vstart
            pltpu.sync_copy(vals_hbm.at[pl.ds(vstart, _CH + 8)], vals_v)
            roff = load_clus(start)

            def node(j, nc):
                prev_c, accs = nc[0], nc[1:]
                gpos = start + j
                p = jnp.minimum(roff + j, 575)
                cl = plsc.load_gather(
                    clus_v, [_splat(p // 8), _splat(p % 8)])
                validv = jnp.full((16,), gpos < s1)
                cl_eff = jnp.where(validv, cl - wid * _CPT, _CPT)
                same = cl_eff == prev_c
                row = jnp.minimum(j + voff, _CH + 7)
                new = []
                for g in range(_HID // 16):
                    v = plsc.load_gather(
                        vals_v, [_splat(row), iota + 16 * g])
                    a = jnp.where(same, jnp.maximum(accs[g], v), v)
                    plsc.store_scatter(slice_v, [cl_eff, iota + 16 * g], a)
                    new.append(a)
                return (cl_eff, *new)

            return lax.fori_loop(0, _CH, node, carry)

        zero = jnp.zeros((16,), jnp.float32)
        lax.fori_loop(0, nchunks, chunk_a,
                      (_splat(_CPT + 1), zero, zero, zero, zero))

        def chunk_b(kk, _):
            start = s0 + kk * _CH
            roff = load_clus(start)

            def grp(j, _):
                p16 = jnp.minimum(roff + 16 * j + iota, 575)
                cl16 = plsc.load_gather(clus_v, [p16 // 8, p16 % 8])
                gpos16 = start + 16 * j + iota
                valid16 = gpos16 < s1
                idxrow = jnp.where(valid16, gpos16, _DUMP)
                plsc.store_scatter(
                    idx_v, [_splat(j // 8), (16 * j) % 128 + iota], idxrow)
                cl_loc = jnp.where(valid16, cl16 - wid * _CPT, _CPT)
                for ch in range(_HID):
                    v = plsc.load_gather(slice_v, [cl_loc, _splat(ch)])
                    plsc.store_scatter(
                        outv_v, [16 * j + iota, _splat(ch)], v)
                return 0

            lax.fori_loop(0, _CH // 16, grp, 0)
            cps = [
                pltpu.async_copy(outv_v.at[pl.ds(128 * sl, 128)],
                                 out_hbm.at[idx_v.at[sl]], sem)
                for sl in range(4)
            ]
            for cp in cps:
                cp.wait()
            return 0

        lax.fori_loop(0, nchunks, chunk_b, 0)

    return k(vals, clus2d, starts)


def _sc_segmax_final(y3, yl, clus2d, starts):
    mesh = plsc.VectorSubcoreMesh(core_axis_name="c", subcore_axis_name="s")

    @functools.partial(
        pl.kernel, mesh=mesh,
        compiler_params=pltpu.CompilerParams(
            needs_layout_passes=False, use_tc_tiling_on_sc=False),
        out_type=jax.ShapeDtypeStruct((_NC, 2 * _HID), jnp.float32),
        scratch_types=[
            pltpu.VMEM((40,), jnp.int32),
            pltpu.VMEM((72, 8), jnp.int32),
            pltpu.VMEM((_CH2 + 8, _HID), jnp.float32),
            pltpu.VMEM((_CH2 + 8, _HID), jnp.float32),
            pltpu.VMEM((_CPT + 1, 2 * _HID), jnp.float32),
            pltpu.SemaphoreType.DMA,
        ],
    )
    def k(y3_hbm, yl_hbm, clus_hbm, starts_hbm, out_hbm,
          starts_v, clus_v, y3_v, yl_v, slice_v, sem):
        iota = lax.iota(jnp.int32, 16)
        wid = lax.axis_index("s") * 2 + lax.axis_index("c")
        pltpu.sync_copy(starts_hbm, starts_v)

        def rd(i):
            return jnp.max(plsc.load_gather(starts_v, [_splat(i)]))

        s0 = rd(wid)
        s1 = rd(wid + 1)
        nchunks = (s1 - s0 + _CH2 - 1) // _CH2

        def init_body(r, _):
            z = jnp.zeros((16,), jnp.float32)
            neg = jnp.full((16,), -3e38, jnp.float32)
            for g in range(_HID // 16):
                slice_v[r, pl.ds(16 * g, 16)] = z
                slice_v[r, pl.ds(_HID + 16 * g, 16)] = neg
            return 0

        lax.fori_loop(0, _CPT + 1, init_body, 0)

        def chunk_a(kk, carry):
            start = s0 + kk * _CH2
            vstart = jnp.minimum(8 * (start // 8), _NN - _CH2 - 8)
            voff = start - vstart
            pltpu.sync_copy(y3_hbm.at[pl.ds(vstart, _CH2 + 8)], y3_v)
            pltpu.sync_copy(yl_hbm.at[pl.ds(vstart, _CH2 + 8)], yl_v)
            crow = jnp.minimum(8 * (start // 64), _CROWS - 72)
            pltpu.sync_copy(clus_hbm.at[pl.ds(crow, 72)], clus_v)
            roff = start - crow * 8

            def node(j, nc):
                prev_c, accs = nc[0], nc[1:]
                gpos = start + j
                p = jnp.minimum(roff + j, 575)
                cl = plsc.load_gather(clus_v, [_splat(p // 8), _splat(p % 8)])
                validv = jnp.full((16,), gpos < s1)
                cl_eff = jnp.where(validv, cl - wid * _CPT, _CPT)
                same = cl_eff == prev_c
                row = jnp.minimum(j + voff, _CH2 + 7)
                new = []
                for g in range(_HID // 16):
                    v = plsc.load_gather(y3_v, [_splat(row), iota + 16 * g])
                    a = jnp.where(same, jnp.maximum(accs[g], v), v)
                    plsc.store_scatter(slice_v, [cl_eff, iota + 16 * g], a)
                    new.append(a)
                for g in range(_HID // 16):
                    v = plsc.load_gather(yl_v, [_splat(row), iota + 16 * g])
                    a = jnp.where(same, jnp.maximum(accs[4 + g], v), v)
                    plsc.store_scatter(
                        slice_v, [cl_eff, _HID + iota + 16 * g], a)
                    new.append(a)
                return (cl_eff, *new)

            return lax.fori_loop(0, _CH2, node, carry)

        zero = jnp.zeros((16,), jnp.float32)
        lax.fori_loop(0, nchunks, chunk_a,
                      (_splat(_CPT + 1),) + (zero,) * 8)
        pltpu.sync_copy(slice_v.at[pl.ds(0, _CPT)],
                        out_hbm.at[pl.ds(wid * _CPT, _CPT)])

    return k(y3, yl, clus2d, starts)


def _seg_gather(vals, clus2d, starts):
    out = _sc_segmax_gather(vals, clus2d, starts)
    return out


def _seg_final(y3, yl, clus2d, starts):
    return _sc_segmax_final(y3, yl, clus2d, starts)


def kernel(x, cluster, edge_index, identifier, valid_len, time_step_len, params):
    del edge_index, time_step_len
    clus2d = cluster.astype(jnp.int32).reshape(_CROWS, 8)
    bnds = jnp.arange(0, _NC + 1, _CPT, dtype=cluster.dtype)
    starts = jnp.searchsorted(cluster, bnds).astype(jnp.int32)
    starts = jnp.concatenate([starts, jnp.full((7,), _NN, jnp.int32)])

    y1 = _layer0(x, params['mlp0'])
    g1 = _seg_gather(y1, clus2d, starts)
    y2 = _layer12(y1, g1, params['mlp1'])
    g2 = _seg_gather(y2, clus2d, starts)
    wla = params['Wl'][:_HID]
    wlb = params['Wl'][_HID:]
    y3, yl = _layer12(y2, g2, params['mlp2'], wla=wla)
    aggcat = _seg_final(y3, yl, clus2d, starts)
    sub = _finalize(aggcat, wlb, params['bl'])

    ones = jnp.ones((_NC, 1), jnp.float32)
    zeros = jnp.zeros((_NC, 128 - _HID - 3), jnp.float32)
    feats = jnp.concatenate([sub, identifier, ones, zeros], axis=1)
    feats = feats.reshape(_B, _T, 128)

    q = params['att0']

    def _pad_w(W, b):
        Wp = jnp.zeros((128, _HID), jnp.float32)
        Wp = Wp.at[:_HID + 2].set(W)
        Wp = Wp.at[_HID + 2].set(b)
        return Wp

    return _attention(feats, valid_len, _pad_w(q['Wq'], q['bq']),
                      _pad_w(q['Wk'], q['bk']), _pad_w(q['Wv'], q['bv']))
